# baseline (speedup 1.0000x reference)
import functools

import jax
import jax.numpy as jnp
from jax import lax
from jax.experimental import pallas as pl
from jax.experimental.pallas import tpu as pltpu
from jax.experimental.pallas import tpu_sc as plsc

SCALE = 0.7071067811865476
EMB = 64
SEQ = 200
PAD = 0


def _sc_embed(tok_flat, word_table, pos_table):
    N = tok_flat.shape[0]
    info = plsc.get_sparse_core_info()
    NW = info.num_cores * info.num_subcores
    per_w = N // NW
    CH = 800
    n_chunks = per_w // CH
    assert per_w % CH == 0 and CH % SEQ == 0

    mesh = plsc.VectorSubcoreMesh(core_axis_name="c", subcore_axis_name="s")

    @functools.partial(
        pl.kernel,
        out_type=jax.ShapeDtypeStruct((N, EMB), jnp.float32),
        mesh=mesh,
        compiler_params=pltpu.CompilerParams(
            needs_layout_passes=False, use_tc_tiling_on_sc=False
        ),
        scratch_types=[
            pltpu.VMEM((SEQ, EMB), jnp.float32),
            pltpu.VMEM((CH,), jnp.int32),
            pltpu.VMEM((CH, EMB), jnp.float32),
            pltpu.SemaphoreType.DMA,
        ],
    )
    def k(tok_hbm, table_hbm, pos_hbm, out_hbm, pos_v, idx_v, rows_v, sem):
        nc = info.num_cores
        wid = lax.axis_index("s") * nc + lax.axis_index("c")

        pltpu.sync_copy(pos_hbm, pos_v)

        def scale_body(i, _):
            r = i // (EMB // 16)
            c16 = (i % (EMB // 16)) * 16
            pos_v[r, pl.ds(c16, 16)] = pos_v[r, pl.ds(c16, 16)] * SCALE
            return 0

        lax.fori_loop(0, SEQ * (EMB // 16), scale_body, 0)

        def chunk_body(c, _):
            base = wid * per_w + c * CH
            pltpu.sync_copy(tok_hbm.at[pl.ds(base, CH)], idx_v)
            pltpu.async_copy(table_hbm.at[idx_v], rows_v, sem).wait()

            def row_body(j, _):
                idxb = plsc.load_gather(idx_v, [jnp.full((16,), j, jnp.int32)])
                m = jnp.where(idxb != PAD, jnp.float32(SCALE), jnp.float32(0.0))
                jm = j % SEQ
                for kk in range(EMB // 16):
                    sl = pl.ds(kk * 16, 16)
                    rows_v[j, sl] = rows_v[j, sl] * m + pos_v[jm, sl]
                return 0

            lax.fori_loop(0, CH, row_body, 0)
            pltpu.sync_copy(rows_v, out_hbm.at[pl.ds(base, CH)])
            return 0

        lax.fori_loop(0, n_chunks, chunk_body, 0)

    return k(tok_flat, word_table, pos_table)


def kernel(tok_ids, word_table, pos_table):
    B, S = tok_ids.shape
    tok_flat = tok_ids.reshape(-1).astype(jnp.int32)
    out = _sc_embed(tok_flat, word_table, pos_table)
    return out.reshape(B, S, EMB)


# speedup vs baseline: 1.0282x; 1.0282x over previous
import functools

import jax
import jax.numpy as jnp
from jax import lax
from jax.experimental import pallas as pl
from jax.experimental.pallas import tpu as pltpu
from jax.experimental.pallas import tpu_sc as plsc

SCALE = 0.7071067811865476
EMB = 64
SEQ = 200
PAD = 0
LANES = 16


def _sc_embed(tok_flat, word_table, pos_table):
    N = tok_flat.shape[0]
    info = plsc.get_sparse_core_info()
    NW = info.num_cores * info.num_subcores
    per_w = N // NW
    CH = 2 * SEQ
    NB = 3
    n_chunks = per_w // CH
    assert per_w % CH == 0
    KV = EMB // LANES

    mesh = plsc.VectorSubcoreMesh(core_axis_name="c", subcore_axis_name="s")

    @functools.partial(
        pl.kernel,
        out_type=jax.ShapeDtypeStruct((N, EMB), jnp.float32),
        mesh=mesh,
        compiler_params=pltpu.CompilerParams(
            needs_layout_passes=False, use_tc_tiling_on_sc=False
        ),
        scratch_types=[
            pltpu.VMEM((SEQ, EMB), jnp.float32),
            pltpu.VMEM((per_w,), jnp.int32),
            pltpu.VMEM((NB, CH, EMB), jnp.float32),
            pltpu.SemaphoreType.DMA((NB,)),
            pltpu.SemaphoreType.DMA((NB,)),
        ],
    )
    def k(tok_hbm, table_hbm, pos_hbm, out_hbm, pos_v, idx_v, rows_v, gsem, wsem):
        nc = info.num_cores
        wid = lax.axis_index("s") * nc + lax.axis_index("c")
        wbase = wid * per_w

        pltpu.sync_copy(tok_hbm.at[pl.ds(wbase, per_w)], idx_v)
        pltpu.sync_copy(pos_hbm, pos_v)

        def scale_body(i, _):
            r = i // KV
            c16 = (i % KV) * LANES
            pos_v[r, pl.ds(c16, 16)] = pos_v[r, pl.ds(c16, 16)] * SCALE
            return 0

        lax.fori_loop(0, SEQ * KV, scale_body, 0)

        def issue_gather(c, b):
            pltpu.async_copy(
                table_hbm.at[idx_v.at[pl.ds(c * CH, CH)]],
                rows_v.at[b],
                gsem.at[b],
            )

        def wait_gather(b):
            pltpu.make_async_copy(
                table_hbm.at[pl.ds(0, CH)], rows_v.at[b], gsem.at[b]
            ).wait()

        def issue_write(c, b):
            pltpu.async_copy(
                rows_v.at[b],
                out_hbm.at[pl.ds(wbase + c * CH, CH)],
                wsem.at[b],
            )

        def wait_write(b):
            pltpu.make_async_copy(
                rows_v.at[b], out_hbm.at[pl.ds(0, CH)], wsem.at[b]
            ).wait()

        issue_gather(0, 0)
        issue_gather(1, 1)

        def chunk_body(c, _):
            b = c % NB
            wait_gather(b)

            cbase = c * CH

            def scan_body(g, acc):
                iv = idx_v[pl.ds(cbase + g * LANES, 16)]
                return acc + jnp.where(iv == PAD, jnp.int32(1), jnp.int32(0))

            acc = lax.fori_loop(
                0, CH // LANES, scan_body, jnp.zeros((16,), jnp.int32)
            )
            npad = jnp.sum(acc)

            @pl.when(npad > 0)
            def _fix():
                def fix_body(j, _):
                    bc = plsc.load_gather(
                        idx_v, [jnp.full((16,), cbase + j, jnp.int32)]
                    )
                    is_pad = bc == PAD
                    for kk in range(KV):
                        sl = pl.ds(kk * LANES, 16)
                        v = rows_v[b, j, sl]
                        rows_v[b, j, sl] = jnp.where(is_pad, jnp.float32(0.0), v)
                    return 0

                lax.fori_loop(0, CH, fix_body, 0)

            def s_body(s, _):
                pvs = [pos_v[s, pl.ds(kk * LANES, 16)] for kk in range(KV)]
                for r in range(CH // SEQ):
                    j = s + r * SEQ
                    for kk in range(KV):
                        sl = pl.ds(kk * LANES, 16)
                        rows_v[b, j, sl] = rows_v[b, j, sl] * SCALE + pvs[kk]
                return 0

            lax.fori_loop(0, SEQ, s_body, 0)

            issue_write(c, b)

            @pl.when(c >= 1)
            def _drain():
                wait_write((c - 1) % NB)

            @pl.when(c + 2 < n_chunks)
            def _next():
                issue_gather(c + 2, (c + 2) % NB)

            return 0

        lax.fori_loop(0, n_chunks, chunk_body, 0)
        wait_write((n_chunks - 1) % NB)

    return k(tok_flat, word_table, pos_table)


def kernel(tok_ids, word_table, pos_table):
    B, S = tok_ids.shape
    tok_flat = tok_ids.reshape(-1).astype(jnp.int32)
    out = _sc_embed(tok_flat, word_table, pos_table)
    return out.reshape(B, S, EMB)
